# Initial kernel scaffold; baseline (speedup 1.0000x reference)
import jax
import jax.numpy as jnp
from jax.experimental import pallas as pl


def kernel(x, hyperedge_index, temb, W_lin, W_temb, b_temb):
    raise NotImplementedError("write your pallas kernel here")


# speedup vs baseline: 6.0876x; 6.0876x over previous
import functools

import jax
import jax.numpy as jnp
from jax import lax
from jax.experimental import pallas as pl
from jax.experimental.pallas import tpu as pltpu
from jax.experimental.pallas import tpu_sc as plsc

N = 10000
NNZ = 160000
F = 256
HALF = 128
R = 10240
W = 144
BLK = 128
NB = 1280
NNZ_PAD = NB * BLK - NNZ
SUBCORES = 16
BPT = NB // SUBCORES
RPT = R // SUBCORES
ZB = 64
IC = 16

_f32 = jnp.float32


def _prep_body(x_ref, wlin_ref, temb_ref, wtemb_ref, btemb_ref,
               xl0_ref, xl1_ref, tproj_ref):
    xl = jnp.dot(x_ref[...], wlin_ref[...].T, preferred_element_type=_f32)
    xl0_ref[...] = jnp.zeros((R, W), _f32)
    xl1_ref[...] = jnp.zeros((R, W), _f32)
    xl0_ref[0:N, 0:HALF] = xl[:, 0:HALF]
    xl1_ref[0:N, 0:HALF] = xl[:, HALF:F]
    xl0_ref[0:N, HALF:HALF + 1] = jnp.ones((N, 1), _f32)
    xl1_ref[0:N, HALF:HALF + 1] = jnp.ones((N, 1), _f32)
    t = temb_ref[...]
    st = t * (1.0 / (1.0 + jnp.exp(-t)))
    tproj_ref[...] = (
        jnp.dot(st, wtemb_ref[...].T, preferred_element_type=_f32)
        + btemb_ref[...])


_prep = pl.pallas_call(
    _prep_body,
    out_shape=(
        jax.ShapeDtypeStruct((R, W), _f32),
        jax.ShapeDtypeStruct((R, W), _f32),
        jax.ShapeDtypeStruct((1, F), _f32),
    ),
)


def _sc_pass_body(src0, src1, gidx, sidx,
                  out0, out1,
                  gloc, sloc, rows, zbuf, acc, sem):
    c = lax.axis_index("c")
    s = lax.axis_index("s")
    rowlo = s * RPT

    def _zrow(i, carry):
        for j in range(W // 16):
            zbuf[i, pl.ds(j * 16, 16)] = jnp.zeros((16,), _f32)
        return carry
    lax.fori_loop(0, ZB, _zrow, 0)

    def _zcp(k, carry):
        pltpu.sync_copy(zbuf, acc.at[pl.ds(rowlo + k * ZB, ZB)])
        return carry
    lax.fori_loop(0, RPT // ZB, _zcp, 0)
    plsc.subcore_barrier()

    def _run(src):
        def _chunk(ch, carry):
            base = s * BPT + ch * IC
            pltpu.sync_copy(gidx.at[pl.ds(base, IC)], gloc)
            pltpu.sync_copy(sidx.at[pl.ds(base, IC)], sloc)

            def _blk(b, c2):
                pltpu.async_copy(src.at[gloc.at[b]], rows, sem).wait()
                pltpu.sync_copy(rows, acc.at[sloc.at[b]], add=True)
                return c2
            lax.fori_loop(0, IC, _blk, 0)
            return carry
        lax.fori_loop(0, BPT // IC, _chunk, 0)

    @pl.when(c == 0)
    def _():
        _run(src0)

    @pl.when(c == 1)
    def _():
        _run(src1)

    plsc.subcore_barrier()

    @pl.when(c == 0)
    def _():
        pltpu.sync_copy(acc.at[pl.ds(rowlo, RPT)], out0.at[pl.ds(rowlo, RPT)])

    @pl.when(c == 1)
    def _():
        pltpu.sync_copy(acc.at[pl.ds(rowlo, RPT)], out1.at[pl.ds(rowlo, RPT)])


_sc_pass = pl.kernel(
    _sc_pass_body,
    out_type=(
        jax.ShapeDtypeStruct((R, W), _f32),
        jax.ShapeDtypeStruct((R, W), _f32),
    ),
    mesh=plsc.VectorSubcoreMesh(core_axis_name="c", subcore_axis_name="s"),
    scratch_types=[
        pltpu.VMEM((IC, BLK), jnp.int32),
        pltpu.VMEM((IC, BLK), jnp.int32),
        pltpu.VMEM((BLK, W), _f32),
        pltpu.VMEM((ZB, W), _f32),
        pltpu.VMEM_SHARED((R, W), _f32),
        pltpu.SemaphoreType.DMA,
    ],
    compiler_params=pltpu.CompilerParams(use_tc_tiling_on_sc=False),
)


def _scale_body(a0_ref, a1_ref, e0_ref, e1_ref):
    a0 = a0_ref[...]
    a1 = a1_ref[...]
    cnt = a0[:, HALF:HALF + 1]
    inv = jnp.where(cnt == 0, 0.0, 1.0 / cnt)
    e0_ref[...] = jnp.zeros((R, W), _f32)
    e1_ref[...] = jnp.zeros((R, W), _f32)
    e0_ref[:, 0:HALF] = a0[:, 0:HALF] * inv
    e1_ref[:, 0:HALF] = a1[:, 0:HALF] * inv
    e0_ref[:, HALF:HALF + 1] = jnp.ones((R, 1), _f32)
    e1_ref[:, HALF:HALF + 1] = jnp.ones((R, 1), _f32)


_scale = pl.pallas_call(
    _scale_body,
    out_shape=(
        jax.ShapeDtypeStruct((R, W), _f32),
        jax.ShapeDtypeStruct((R, W), _f32),
    ),
)


def _final_body(b0_ref, b1_ref, tproj_ref, out_ref):
    b0 = b0_ref[...]
    b1 = b1_ref[...]
    cnt = b0[0:N, HALF:HALF + 1]
    inv = jnp.where(cnt == 0, 0.0, 1.0 / cnt)
    node_out = jnp.concatenate(
        [b0[0:N, 0:HALF] * inv, b1[0:N, 0:HALF] * inv], axis=1)
    h = node_out + tproj_ref[...]
    out_ref[...] = h * (1.0 / (1.0 + jnp.exp(-h)))


_final = pl.pallas_call(
    _final_body,
    out_shape=jax.ShapeDtypeStruct((N, F), _f32),
)


def kernel(x, hyperedge_index, temb, W_lin, W_temb, b_temb):
    pad = jnp.full((NNZ_PAD,), N, jnp.int32)
    nidx = jnp.concatenate([hyperedge_index[0], pad]).reshape(NB, BLK)
    eidx = jnp.concatenate([hyperedge_index[1], pad]).reshape(NB, BLK)

    xl0, xl1, tproj = _prep(x, W_lin, temb, W_temb, b_temb.reshape(1, F))
    a0, a1 = _sc_pass(xl0, xl1, nidx, eidx)
    ef0, ef1 = _scale(a0, a1)
    b0, b1 = _sc_pass(ef0, ef1, eidx, nidx)
    return _final(b0, b1, tproj)


# speedup vs baseline: 7.2414x; 1.1895x over previous
import jax
import jax.numpy as jnp
from jax import lax
from jax.experimental import pallas as pl
from jax.experimental.pallas import tpu as pltpu
from jax.experimental.pallas import tpu_sc as plsc

N = 10000
NNZ = 160000
F = 256
HALF = 128
R = 10240
BLK = 128
NB = 1280
NNZ_PAD = NB * BLK - NNZ
SUBCORES = 16
BPT = NB // SUBCORES
IC = 16
NCH = BPT // IC
RPT = R // SUBCORES
ZB = 32
ZC = 64
CW = 8

_f32 = jnp.float32


def _prep_body(x_ref, wlin_ref, temb_ref, wtemb_ref, btemb_ref,
               xl0_ref, xl1_ref, tproj_ref):
    xl = jnp.dot(x_ref[...], wlin_ref[...].T, preferred_element_type=_f32)
    xl0_ref[...] = jnp.zeros((R, HALF), _f32)
    xl1_ref[...] = jnp.zeros((R, HALF), _f32)
    xl0_ref[0:N, :] = xl[:, 0:HALF]
    xl1_ref[0:N, :] = xl[:, HALF:F]
    t = temb_ref[...]
    st = t * (1.0 / (1.0 + jnp.exp(-t)))
    tproj_ref[...] = (
        jnp.dot(st, wtemb_ref[...].T, preferred_element_type=_f32)
        + btemb_ref[...])


_prep = pl.pallas_call(
    _prep_body,
    out_shape=(
        jax.ShapeDtypeStruct((R, HALF), _f32),
        jax.ShapeDtypeStruct((R, HALF), _f32),
        jax.ShapeDtypeStruct((1, F), _f32),
    ),
)


def _sc_pass_body(src0, src1, gidx, sidx, ones_h, z128_h, z8_h,
                  out0, out1, cs_out, cg_out,
                  gloc, sloc, rows0, rows1, onesb, zbuf, zcnt, acc, cnt,
                  sem0, sem1):
    c = lax.axis_index("c")
    s = lax.axis_index("s")
    rowlo = s * RPT

    pltpu.sync_copy(z128_h, zbuf)
    pltpu.sync_copy(z8_h, zcnt)
    pltpu.sync_copy(ones_h, onesb)

    def _za(k, carry):
        pltpu.sync_copy(zbuf, acc.at[pl.ds(rowlo + k * ZB, ZB)])
        return carry
    lax.fori_loop(0, RPT // ZB, _za, 0)

    def _zc(k, carry):
        pltpu.sync_copy(zcnt, cnt.at[pl.ds(rowlo + k * ZC, ZC)])
        return carry
    lax.fori_loop(0, RPT // ZC, _zc, 0)
    plsc.subcore_barrier()

    def _run(src, cidx):
        def _chunk(ch, carry):
            base = s * BPT + ch * IC
            pltpu.sync_copy(gidx.at[pl.ds(base, IC)], gloc)
            pltpu.sync_copy(sidx.at[pl.ds(base, IC)], sloc)
            pltpu.async_copy(src.at[gloc.at[0]], rows0, sem0)

            def _pair(k, c2):
                b0 = 2 * k
                b1 = 2 * k + 1
                pltpu.async_copy(src.at[gloc.at[b1]], rows1, sem1)
                pltpu.make_async_copy(src.at[gloc.at[b0]], rows0, sem0).wait()
                pltpu.sync_copy(rows0, acc.at[sloc.at[b0]], add=True)
                pltpu.sync_copy(onesb, cnt.at[cidx.at[b0]], add=True)

                @pl.when(k < IC // 2 - 1)
                def _():
                    pltpu.async_copy(src.at[gloc.at[b0 + 2]], rows0, sem0)

                pltpu.make_async_copy(src.at[gloc.at[b1]], rows1, sem1).wait()
                pltpu.sync_copy(rows1, acc.at[sloc.at[b1]], add=True)
                pltpu.sync_copy(onesb, cnt.at[cidx.at[b1]], add=True)
                return c2
            lax.fori_loop(0, IC // 2, _pair, 0)
            return carry
        lax.fori_loop(0, NCH, _chunk, 0)

    @pl.when(c == 0)
    def _():
        _run(src0, sloc)

    @pl.when(c == 1)
    def _():
        _run(src1, gloc)

    plsc.subcore_barrier()

    @pl.when(c == 0)
    def _():
        pltpu.sync_copy(acc.at[pl.ds(rowlo, RPT)], out0.at[pl.ds(rowlo, RPT)])
        pltpu.sync_copy(cnt.at[pl.ds(rowlo, RPT)], cs_out.at[pl.ds(rowlo, RPT)])

    @pl.when(c == 1)
    def _():
        pltpu.sync_copy(acc.at[pl.ds(rowlo, RPT)], out1.at[pl.ds(rowlo, RPT)])
        pltpu.sync_copy(cnt.at[pl.ds(rowlo, RPT)], cg_out.at[pl.ds(rowlo, RPT)])


_sc_pass = pl.kernel(
    _sc_pass_body,
    out_type=(
        jax.ShapeDtypeStruct((R, HALF), _f32),
        jax.ShapeDtypeStruct((R, HALF), _f32),
        jax.ShapeDtypeStruct((R, CW), _f32),
        jax.ShapeDtypeStruct((R, CW), _f32),
    ),
    mesh=plsc.VectorSubcoreMesh(core_axis_name="c", subcore_axis_name="s"),
    scratch_types=[
        pltpu.VMEM((IC, BLK), jnp.int32),
        pltpu.VMEM((IC, BLK), jnp.int32),
        pltpu.VMEM((BLK, HALF), _f32),
        pltpu.VMEM((BLK, HALF), _f32),
        pltpu.VMEM((BLK, CW), _f32),
        pltpu.VMEM((ZB, HALF), _f32),
        pltpu.VMEM((ZC, CW), _f32),
        pltpu.VMEM_SHARED((R, HALF), _f32),
        pltpu.VMEM_SHARED((R, CW), _f32),
        pltpu.SemaphoreType.DMA,
        pltpu.SemaphoreType.DMA,
    ],
    compiler_params=pltpu.CompilerParams(use_tc_tiling_on_sc=False),
)


def _scale_body(a0_ref, a1_ref, cnt_ref, e0_ref, e1_ref):
    cnt = cnt_ref[...][:, 0:1]
    inv = jnp.where(cnt == 0, 0.0, 1.0 / cnt)
    e0_ref[...] = a0_ref[...] * inv
    e1_ref[...] = a1_ref[...] * inv


_scale = pl.pallas_call(
    _scale_body,
    out_shape=(
        jax.ShapeDtypeStruct((R, HALF), _f32),
        jax.ShapeDtypeStruct((R, HALF), _f32),
    ),
)


def _final_body(b0_ref, b1_ref, cnt_ref, tproj_ref, out_ref):
    cnt = cnt_ref[0:N, 0:1]
    inv = jnp.where(cnt == 0, 0.0, 1.0 / cnt)
    node_out = jnp.concatenate(
        [b0_ref[0:N, :] * inv, b1_ref[0:N, :] * inv], axis=1)
    h = node_out + tproj_ref[...]
    out_ref[...] = h * (1.0 / (1.0 + jnp.exp(-h)))


_final = pl.pallas_call(
    _final_body,
    out_shape=jax.ShapeDtypeStruct((N, F), _f32),
)


def kernel(x, hyperedge_index, temb, W_lin, W_temb, b_temb):
    pad = jnp.full((NNZ_PAD,), N, jnp.int32)
    nidx = jnp.concatenate([hyperedge_index[0], pad]).reshape(NB, BLK)
    eidx = jnp.concatenate([hyperedge_index[1], pad]).reshape(NB, BLK)
    ones_h = jnp.ones((BLK, CW), _f32)
    z128 = jnp.zeros((ZB, HALF), _f32)
    z8 = jnp.zeros((ZC, CW), _f32)

    xl0, xl1, tproj = _prep(x, W_lin, temb, W_temb, b_temb.reshape(1, F))
    a0, a1, cntB, cntD = _sc_pass(xl0, xl1, nidx, eidx, ones_h, z128, z8)
    ef0, ef1 = _scale(a0, a1, cntB)
    b0, b1, _, _ = _sc_pass(ef0, ef1, eidx, nidx, ones_h, z128, z8)
    return _final(b0, b1, cntD, tproj)


# speedup vs baseline: 7.4329x; 1.0264x over previous
import jax
import jax.numpy as jnp
from jax import lax
from jax.experimental import pallas as pl
from jax.experimental.pallas import tpu as pltpu
from jax.experimental.pallas import tpu_sc as plsc

N = 10000
NNZ = 160000
F = 256
HALF = 128
R = 10240
BLK = 64
NB = 2560
NNZ_PAD = NB * BLK - NNZ
SUBCORES = 16
BPT = NB // SUBCORES
IC = 32
NCH = BPT // IC
DEPTH = 4
RPT = R // SUBCORES
ZB = 32
ZC = 64
CW = 8

_f32 = jnp.float32


def _prep_body(x_ref, wlin_ref, temb_ref, wtemb_ref, btemb_ref,
               xl0_ref, xl1_ref, tproj_ref):
    xl = jnp.dot(x_ref[...], wlin_ref[...].T, preferred_element_type=_f32)
    xl0_ref[...] = jnp.zeros((R, HALF), _f32)
    xl1_ref[...] = jnp.zeros((R, HALF), _f32)
    xl0_ref[0:N, :] = xl[:, 0:HALF]
    xl1_ref[0:N, :] = xl[:, HALF:F]
    t = temb_ref[...]
    st = t * (1.0 / (1.0 + jnp.exp(-t)))
    tproj_ref[...] = (
        jnp.dot(st, wtemb_ref[...].T, preferred_element_type=_f32)
        + btemb_ref[...])


_prep = pl.pallas_call(
    _prep_body,
    out_shape=(
        jax.ShapeDtypeStruct((R, HALF), _f32),
        jax.ShapeDtypeStruct((R, HALF), _f32),
        jax.ShapeDtypeStruct((1, F), _f32),
    ),
)


def _make_sc_pass(with_counts):
    def body(src0, src1, gidx, sidx, ones_h, z128_h, z8_h,
             out0, out1, cs_out, cg_out,
             gloc, sloc, r0, r1, r2, r3, onesb, zbuf, zcnt, acc, cnt,
             s0, s1, s2, s3):
        rows = (r0, r1, r2, r3)
        sems = (s0, s1, s2, s3)
        c = lax.axis_index("c")
        s = lax.axis_index("s")
        rowlo = s * RPT

        pltpu.sync_copy(z128_h, zbuf)

        def _za(k, carry):
            pltpu.sync_copy(zbuf, acc.at[pl.ds(rowlo + k * ZB, ZB)])
            return carry
        lax.fori_loop(0, RPT // ZB, _za, 0)

        if with_counts:
            pltpu.sync_copy(z8_h, zcnt)
            pltpu.sync_copy(ones_h, onesb)

            def _zc(k, carry):
                pltpu.sync_copy(zcnt, cnt.at[pl.ds(rowlo + k * ZC, ZC)])
                return carry
            lax.fori_loop(0, RPT // ZC, _zc, 0)
        plsc.subcore_barrier()

        def _run(src, cidx):
            def _chunk(ch, carry):
                base = s * BPT + ch * IC
                pltpu.sync_copy(gidx.at[pl.ds(base, IC)], gloc)
                pltpu.sync_copy(sidx.at[pl.ds(base, IC)], sloc)
                for j in range(DEPTH):
                    pltpu.async_copy(src.at[gloc.at[j]], rows[j], sems[j])

                def _grp(g, c2):
                    for j in range(DEPTH):
                        b = DEPTH * g + j
                        pltpu.make_async_copy(
                            src.at[gloc.at[b]], rows[j], sems[j]).wait()
                        pltpu.sync_copy(
                            rows[j], acc.at[sloc.at[b]], add=True)
                        if with_counts:
                            pltpu.sync_copy(
                                onesb, cnt.at[cidx.at[b]], add=True)

                        @pl.when(b + DEPTH < IC)
                        def _():
                            pltpu.async_copy(
                                src.at[gloc.at[b + DEPTH]], rows[j], sems[j])
                    return c2
                lax.fori_loop(0, IC // DEPTH, _grp, 0)
                return carry
            lax.fori_loop(0, NCH, _chunk, 0)

        @pl.when(c == 0)
        def _():
            _run(src0, sloc)

        @pl.when(c == 1)
        def _():
            _run(src1, gloc)

        plsc.subcore_barrier()
        sl = pl.ds(rowlo, RPT)

        @pl.when(c == 0)
        def _():
            pltpu.sync_copy(acc.at[sl], out0.at[sl])
            if with_counts:
                pltpu.sync_copy(cnt.at[sl], cs_out.at[sl])

        @pl.when(c == 1)
        def _():
            pltpu.sync_copy(acc.at[sl], out1.at[sl])
            if with_counts:
                pltpu.sync_copy(cnt.at[sl], cg_out.at[sl])

    return pl.kernel(
        body,
        out_type=(
            jax.ShapeDtypeStruct((R, HALF), _f32),
            jax.ShapeDtypeStruct((R, HALF), _f32),
            jax.ShapeDtypeStruct((R, CW), _f32),
            jax.ShapeDtypeStruct((R, CW), _f32),
        ),
        mesh=plsc.VectorSubcoreMesh(core_axis_name="c", subcore_axis_name="s"),
        scratch_types=[
            pltpu.VMEM((IC, BLK), jnp.int32),
            pltpu.VMEM((IC, BLK), jnp.int32),
            pltpu.VMEM((BLK, HALF), _f32),
            pltpu.VMEM((BLK, HALF), _f32),
            pltpu.VMEM((BLK, HALF), _f32),
            pltpu.VMEM((BLK, HALF), _f32),
            pltpu.VMEM((BLK, CW), _f32),
            pltpu.VMEM((ZB, HALF), _f32),
            pltpu.VMEM((ZC, CW), _f32),
            pltpu.VMEM_SHARED((R, HALF), _f32),
            pltpu.VMEM_SHARED((R, CW), _f32),
            pltpu.SemaphoreType.DMA,
            pltpu.SemaphoreType.DMA,
            pltpu.SemaphoreType.DMA,
            pltpu.SemaphoreType.DMA,
        ],
        compiler_params=pltpu.CompilerParams(use_tc_tiling_on_sc=False),
    )


_sc_pass1 = _make_sc_pass(with_counts=True)
_sc_pass2 = _make_sc_pass(with_counts=False)


def _scale_body(a0_ref, a1_ref, cnt_ref, e0_ref, e1_ref):
    cnt = cnt_ref[...][:, 0:1]
    inv = jnp.where(cnt == 0, 0.0, 1.0 / cnt)
    e0_ref[...] = a0_ref[...] * inv
    e1_ref[...] = a1_ref[...] * inv


_scale = pl.pallas_call(
    _scale_body,
    out_shape=(
        jax.ShapeDtypeStruct((R, HALF), _f32),
        jax.ShapeDtypeStruct((R, HALF), _f32),
    ),
)


def _final_body(b0_ref, b1_ref, cnt_ref, tproj_ref, out_ref):
    cnt = cnt_ref[0:N, 0:1]
    inv = jnp.where(cnt == 0, 0.0, 1.0 / cnt)
    node_out = jnp.concatenate(
        [b0_ref[0:N, :] * inv, b1_ref[0:N, :] * inv], axis=1)
    h = node_out + tproj_ref[...]
    out_ref[...] = h * (1.0 / (1.0 + jnp.exp(-h)))


_final = pl.pallas_call(
    _final_body,
    out_shape=jax.ShapeDtypeStruct((N, F), _f32),
)


def kernel(x, hyperedge_index, temb, W_lin, W_temb, b_temb):
    pad = jnp.full((NNZ_PAD,), N, jnp.int32)
    nidx = jnp.concatenate([hyperedge_index[0], pad]).reshape(NB, BLK)
    eidx = jnp.concatenate([hyperedge_index[1], pad]).reshape(NB, BLK)
    ones_h = jnp.ones((BLK, CW), _f32)
    z128 = jnp.zeros((ZB, HALF), _f32)
    z8 = jnp.zeros((ZC, CW), _f32)

    xl0, xl1, tproj = _prep(x, W_lin, temb, W_temb, b_temb.reshape(1, F))
    a0, a1, cntB, cntD = _sc_pass1(xl0, xl1, nidx, eidx, ones_h, z128, z8)
    ef0, ef1 = _scale(a0, a1, cntB)
    b0, b1, _, _ = _sc_pass2(ef0, ef1, eidx, nidx, ones_h, z128, z8)
    return _final(b0, b1, cntD, tproj)
